# baseline (speedup 1.0000x reference)
import functools

import jax
import jax.numpy as jnp
from jax import lax
from jax.experimental import pallas as pl
from jax.experimental.pallas import tpu as pltpu
from jax.experimental.pallas import tpu_sc as plsc

CH, CW = 16, 16


def kernel(feature, ROIs):
    B, H, W, C = feature.shape
    N = ROIs.shape[0]
    NW = 32
    per_w = (N + NW - 1) // NW

    rois_flat = jnp.pad(
        ROIs.reshape(N, 6), ((0, NW * per_w + 1 - N), (0, 2))
    ).reshape(-1)

    mesh = plsc.VectorSubcoreMesh(
        core_axis_name="c", subcore_axis_name="s", num_cores=2, num_subcores=16
    )

    @functools.partial(
        pl.kernel,
        out_type=jax.ShapeDtypeStruct((N, CH, CW, C), jnp.float32),
        mesh=mesh,
        compiler_params=pltpu.CompilerParams(use_tc_tiling_on_sc=True),
        scratch_types=[
            pltpu.VMEM(((per_w + 1) * 8,), jnp.int32),
            pltpu.VMEM((16,), jnp.int32),
            pltpu.VMEM((2, 8, 24, C), jnp.float32),
            pltpu.VMEM((2, CH, CW, C), jnp.float32),
            pltpu.SemaphoreType.DMA,
            pltpu.SemaphoreType.DMA,
            pltpu.SemaphoreType.DMA,
            pltpu.SemaphoreType.DMA,
        ],
    )
    def _crop(feat_hbm, rois_hbm, out_hbm, rois_v, civ_v, wbuf, crop,
              semw0, semw1, semo0, semo1):
        wid = lax.axis_index("s") * 2 + lax.axis_index("c")
        start = wid * per_w
        pltpu.sync_copy(rois_hbm.at[pl.ds(start * 8, (per_w + 1) * 8)], rois_v)
        cnt = jnp.maximum(0, jnp.minimum(per_w, N - start))
        semw = (semw0, semw1)
        semo = (semo0, semo1)

        lane = lax.iota(jnp.int32, 16)
        zeros16 = jnp.zeros((16,), jnp.float32)

        def issue_half(b, y1, wa, yint, h, wb):
            @pl.when(yint)
            def _fast():
                pltpu.async_copy(
                    feat_hbm.at[b, pl.ds(y1 + 8 * h, 8), pl.ds(wa, 24)],
                    wbuf.at[wb], semw[wb],
                )

            @pl.when(jnp.logical_not(yint))
            def _slow():
                for r in range(8):
                    rc = jnp.clip(y1 + h * 8 + r, 0, H - 1)
                    pltpu.async_copy(
                        feat_hbm.at[b, rc, pl.ds(wa, 24)], wbuf.at[wb, r],
                        semw[wb],
                    )

        def drain_half(b, y1, wa, yint, h, wb):
            @pl.when(yint)
            def _fast():
                pltpu.make_async_copy(
                    feat_hbm.at[b, pl.ds(y1 + 8 * h, 8), pl.ds(wa, 24)],
                    wbuf.at[wb], semw[wb],
                ).wait()

            @pl.when(jnp.logical_not(yint))
            def _slow():
                for r in range(8):
                    rc = jnp.clip(y1 + h * 8 + r, 0, H - 1)
                    pltpu.make_async_copy(
                        feat_hbm.at[b, rc, pl.ds(wa, 24)], wbuf.at[wb, r],
                        semw[wb],
                    ).wait()

        def coords(t):
            rec = rois_v[pl.ds(t * 8, 16)]
            return rec[0], rec[1], rec[2]

        def step(t, buf):
            n = start + t
            b, y1, x1 = coords(t)
            xs = jnp.clip(x1, 0, W - CW)
            wa = jnp.minimum((xs // 8) * 8, W - 24)
            cols = x1 + lane
            civ_v[...] = jnp.where((cols < 0) | (cols >= W), 1, 0)
            yint = (y1 >= 0) & (y1 <= H - CH)
            interior = yint & (x1 >= 0) & (x1 <= W - CW)
            ojs = [jnp.clip(x1 + j - wa, 0, 23) for j in range(CW)]
            bn, y1n, x1n = coords(t + 1)
            xsn = jnp.clip(x1n, 0, W - CW)
            wan = jnp.minimum((xsn // 8) * 8, W - 24)
            yintn = (y1n >= 0) & (y1n <= H - CH)

            @pl.when(t >= 2)
            def _wo():
                pltpu.make_async_copy(
                    crop.at[buf], out_hbm.at[n - 2], semo[buf]
                ).wait()

            for h in range(2):
                drain_half(b, y1, wa, yint, h, h)

                for j in range(CW):
                    def repack(r, cr, h=h, j=j):
                        i = h * 8 + r
                        for v in range(C // 16):
                            crop[buf, i, j, pl.ds(v * 16, 16)] = (
                                wbuf[h, r, ojs[j], pl.ds(v * 16, 16)]
                            )
                        return cr

                    lax.fori_loop(0, 8, repack, 0)

                @pl.when(t + 1 < cnt)
                def _refill(h=h):
                    issue_half(bn, y1n, wan, yintn, h, h)

            @pl.when(jnp.logical_not(interior))
            def _zero_oob():
                civ = civ_v[...]

                def zrow(i, cr):
                    y = y1 + i
                    rinv = ((y < 0) | (y >= H)).astype(jnp.int32)
                    for j in range(CW):
                        @pl.when((rinv + civ[j]) > 0)
                        def _z(j=j):
                            for v in range(C // 16):
                                crop[buf, i, j, pl.ds(v * 16, 16)] = zeros16
                    return cr

                lax.fori_loop(0, CH, zrow, 0)

            pltpu.async_copy(crop.at[buf], out_hbm.at[n], semo[buf])

        @pl.when(cnt > 0)
        def _run():
            b0, y10, x10 = coords(0)
            xs0 = jnp.clip(x10, 0, W - CW)
            wa0 = jnp.minimum((xs0 // 8) * 8, W - 24)
            yint0 = (y10 >= 0) & (y10 <= H - CH)
            issue_half(b0, y10, wa0, yint0, 0, 0)
            issue_half(b0, y10, wa0, yint0, 1, 1)

            def outer(p, carry):
                t0 = 2 * p
                step(t0, 0)

                @pl.when(t0 + 1 < cnt)
                def _s1():
                    step(t0 + 1, 1)

                return carry

            lax.fori_loop(0, (cnt + 1) // 2, outer, 0)
            pltpu.make_async_copy(
                crop.at[0], out_hbm.at[start + cnt - 2], semo[0]
            ).wait()

            @pl.when(cnt >= 2)
            def _wlast():
                pltpu.make_async_copy(
                    crop.at[1], out_hbm.at[start + cnt - 1], semo[1]
                ).wait()

    out = _crop(feature, rois_flat)
    return out


# speedup vs baseline: 1.1519x; 1.1519x over previous
import functools

import jax
import jax.numpy as jnp
from jax import lax
from jax.experimental import pallas as pl
from jax.experimental.pallas import tpu as pltpu
from jax.experimental.pallas import tpu_sc as plsc

CH, CW = 16, 16


def kernel(feature, ROIs):
    B, H, W, C = feature.shape
    N = ROIs.shape[0]
    NW = 32
    per_w = (N + NW - 1) // NW

    rois_flat = jnp.pad(
        ROIs.reshape(N, 6), ((0, NW * per_w + 1 - N), (0, 2))
    ).reshape(-1)

    mesh = plsc.VectorSubcoreMesh(
        core_axis_name="c", subcore_axis_name="s", num_cores=2, num_subcores=16
    )

    @functools.partial(
        pl.kernel,
        out_type=jax.ShapeDtypeStruct((N, CH, CW, C), jnp.float32),
        mesh=mesh,
        compiler_params=pltpu.CompilerParams(use_tc_tiling_on_sc=True),
        scratch_types=[
            pltpu.VMEM(((per_w + 1) * 8,), jnp.int32),
            pltpu.VMEM((16,), jnp.int32),
            pltpu.VMEM((2, 8, 24, C), jnp.float32),
            pltpu.VMEM((2, CH, CW, C), jnp.float32),
            pltpu.SemaphoreType.DMA,
            pltpu.SemaphoreType.DMA,
            pltpu.SemaphoreType.DMA,
            pltpu.SemaphoreType.DMA,
        ],
    )
    def _crop(feat_hbm, rois_hbm, out_hbm, rois_v, civ_v, wbuf, crop,
              semw0, semw1, semo0, semo1):
        wid = lax.axis_index("s") * 2 + lax.axis_index("c")
        start = wid * per_w
        pltpu.sync_copy(rois_hbm.at[pl.ds(start * 8, (per_w + 1) * 8)], rois_v)
        cnt = jnp.maximum(0, jnp.minimum(per_w, N - start))
        semw = (semw0, semw1)
        semo = (semo0, semo1)

        lane = lax.iota(jnp.int32, 16)
        zeros16 = jnp.zeros((16,), jnp.float32)

        def issue_half(b, y1, wa, yint, h, wb):
            @pl.when(yint)
            def _fast():
                pltpu.async_copy(
                    feat_hbm.at[b, pl.ds(y1 + 8 * h, 8), pl.ds(wa, 24)],
                    wbuf.at[wb], semw[wb],
                )

            @pl.when(jnp.logical_not(yint))
            def _slow():
                for r in range(8):
                    rc = jnp.clip(y1 + h * 8 + r, 0, H - 1)
                    pltpu.async_copy(
                        feat_hbm.at[b, rc, pl.ds(wa, 24)], wbuf.at[wb, r],
                        semw[wb],
                    )

        def drain_half(b, y1, wa, yint, h, wb):
            @pl.when(yint)
            def _fast():
                pltpu.make_async_copy(
                    feat_hbm.at[b, pl.ds(y1 + 8 * h, 8), pl.ds(wa, 24)],
                    wbuf.at[wb], semw[wb],
                ).wait()

            @pl.when(jnp.logical_not(yint))
            def _slow():
                for r in range(8):
                    rc = jnp.clip(y1 + h * 8 + r, 0, H - 1)
                    pltpu.make_async_copy(
                        feat_hbm.at[b, rc, pl.ds(wa, 24)], wbuf.at[wb, r],
                        semw[wb],
                    ).wait()

        def coords(t):
            rec = rois_v[pl.ds(t * 8, 16)]
            return rec[0], rec[1], rec[2]

        def step(t, buf):
            n = start + t
            b, y1, x1 = coords(t)
            xs = jnp.clip(x1, 0, W - CW)
            wa = jnp.minimum((xs // 8) * 8, W - 24)
            cols = x1 + lane
            civ_v[...] = jnp.where((cols < 0) | (cols >= W), 1, 0)
            yint = (y1 >= 0) & (y1 <= H - CH)
            interior = yint & (x1 >= 0) & (x1 <= W - CW)
            ojs = [jnp.clip(x1 + j - wa, 0, 23) for j in range(CW)]
            bn, y1n, x1n = coords(t + 1)
            xsn = jnp.clip(x1n, 0, W - CW)
            wan = jnp.minimum((xsn // 8) * 8, W - 24)
            yintn = (y1n >= 0) & (y1n <= H - CH)

            @pl.when(t >= 2)
            def _wo():
                pltpu.make_async_copy(
                    crop.at[buf], out_hbm.at[n - 2], semo[buf]
                ).wait()

            for h in range(2):
                drain_half(b, y1, wa, yint, h, h)

                def repack(r, cr, h=h):
                    i = h * 8 + r
                    for j in range(CW):
                        for v in range(C // 16):
                            crop[buf, i, j, pl.ds(v * 16, 16)] = (
                                wbuf[h, r, ojs[j], pl.ds(v * 16, 16)]
                            )
                    return cr

                lax.fori_loop(0, 8, repack, 0)

                @pl.when(t + 1 < cnt)
                def _refill(h=h):
                    issue_half(bn, y1n, wan, yintn, h, h)

            @pl.when(jnp.logical_not(interior))
            def _zero_oob():
                civ = civ_v[...]

                def zrow(i, cr):
                    y = y1 + i
                    rinv = ((y < 0) | (y >= H)).astype(jnp.int32)
                    for j in range(CW):
                        @pl.when((rinv + civ[j]) > 0)
                        def _z(j=j):
                            for v in range(C // 16):
                                crop[buf, i, j, pl.ds(v * 16, 16)] = zeros16
                    return cr

                lax.fori_loop(0, CH, zrow, 0)

            pltpu.async_copy(crop.at[buf], out_hbm.at[n], semo[buf])

        @pl.when(cnt > 0)
        def _run():
            b0, y10, x10 = coords(0)
            xs0 = jnp.clip(x10, 0, W - CW)
            wa0 = jnp.minimum((xs0 // 8) * 8, W - 24)
            yint0 = (y10 >= 0) & (y10 <= H - CH)
            issue_half(b0, y10, wa0, yint0, 0, 0)
            issue_half(b0, y10, wa0, yint0, 1, 1)

            def outer(p, carry):
                t0 = 2 * p
                step(t0, 0)

                @pl.when(t0 + 1 < cnt)
                def _s1():
                    step(t0 + 1, 1)

                return carry

            lax.fori_loop(0, (cnt + 1) // 2, outer, 0)
            pltpu.make_async_copy(
                crop.at[0], out_hbm.at[start + cnt - 2], semo[0]
            ).wait()

            @pl.when(cnt >= 2)
            def _wlast():
                pltpu.make_async_copy(
                    crop.at[1], out_hbm.at[start + cnt - 1], semo[1]
                ).wait()

    out = _crop(feature, rois_flat)
    return out
